# Initial kernel scaffold; baseline (speedup 1.0000x reference)
import jax
import jax.numpy as jnp
from jax.experimental import pallas as pl


def kernel(mailbox_m):
    raise NotImplementedError("write your pallas kernel here")


# speedup vs baseline: 1.1264x; 1.1264x over previous
import jax
import jax.numpy as jnp
from jax.experimental import pallas as pl

N_NODES = 10000
MAX_DEG = 32
D_FEAT = 128
BN = 400


def _mean_body(x_ref, o_ref):
    o_ref[...] = jnp.sum(x_ref[...], axis=1) * (1.0 / MAX_DEG)


def kernel(mailbox_m):
    grid = (N_NODES // BN,)
    return pl.pallas_call(
        _mean_body,
        grid=grid,
        in_specs=[pl.BlockSpec((BN, MAX_DEG, D_FEAT), lambda i: (i, 0, 0))],
        out_specs=pl.BlockSpec((BN, D_FEAT), lambda i: (i, 0)),
        out_shape=jax.ShapeDtypeStruct((N_NODES, D_FEAT), jnp.float32),
    )(mailbox_m)
